# Initial kernel scaffold; baseline (speedup 1.0000x reference)
import jax
import jax.numpy as jnp
from jax.experimental import pallas as pl


def kernel(input_coords, input_x, anchor_coords, flow_dir, params):
    raise NotImplementedError("write your pallas kernel here")


# speedup vs baseline: 7.6424x; 7.6424x over previous
import functools

import jax
import jax.numpy as jnp
from jax import lax
from jax.experimental import pallas as pl
from jax.experimental.pallas import tpu as pltpu
from jax.experimental.pallas import tpu_sc as plsc

N = 100000
A = 512
K = 128
IN_CH = 16
HID = 64
TBL_W = 128
C_SEL = 2048
N_PAD = 100352
N_BLK = 2000
AB = 32

_SQRT_HALF = 0.7071067811865476


def _gelu(x):
    return 0.5 * x * (1.0 + lax.erf(x * _SQRT_HALF))


def _q(x):
    return x.astype(jnp.bfloat16).astype(jnp.float32)


def _dot_bf(x, w):
    return jnp.dot(x.astype(jnp.bfloat16), w.astype(jnp.bfloat16),
                   preferred_element_type=jnp.float32)


def _mlp3(x, w0, b0, w1, b1, w2, b2):
    h = _gelu(_dot_bf(x, w0) + b0)
    h = _gelu(_dot_bf(h, w1) + b1)
    return _dot_bf(h, w2) + b2


def _prep_body(x_ref, fin_ref, u_ref, w0, b0, w1, b1, w2, b2,
               tbl_ref, fsum_ref):
    feat = _mlp3(fin_ref[...], w0[...], b0[...], w1[...], b1[...],
                 w2[...], b2[...])
    xq = _q(x_ref[...])
    uq = _q(u_ref[...])
    pq = xq * uq
    ndu_bf = pq[:, 0:1] + pq[:, 1:2] + pq[:, 2:3]
    pe = x_ref[...] * u_ref[...]
    ndu_ex = pe[:, 0:1] + pe[:, 1:2] + pe[:, 2:3]
    pad = jnp.zeros((feat.shape[0], TBL_W - HID - 2), jnp.float32)
    tbl_ref[...] = jnp.concatenate([feat, ndu_bf, ndu_ex, pad], axis=1)
    part = jnp.sum(feat, axis=0, keepdims=True)

    @pl.when(pl.program_id(0) == 0)
    def _():
        fsum_ref[...] = part

    @pl.when(pl.program_id(0) != 0)
    def _():
        fsum_ref[...] += part


def _run_prep(x, fin, u_row, p):
    grid = N // N_BLK
    full = lambda shape: pl.BlockSpec(shape, lambda i: (0, 0))
    return pl.pallas_call(
        _prep_body,
        grid=(grid,),
        in_specs=[
            pl.BlockSpec((N_BLK, 3), lambda i: (i, 0)),
            pl.BlockSpec((N_BLK, IN_CH), lambda i: (i, 0)),
            full((1, 3)),
            full((IN_CH, HID)), full((1, HID)),
            full((HID, HID)), full((1, HID)),
            full((HID, HID)), full((1, HID)),
        ],
        out_specs=[
            pl.BlockSpec((N_BLK, TBL_W), lambda i: (i, 0)),
            pl.BlockSpec((1, HID), lambda i: (0, 0)),
        ],
        out_shape=[
            jax.ShapeDtypeStruct((N, TBL_W), jnp.float32),
            jax.ShapeDtypeStruct((1, HID), jnp.float32),
        ],
    )(x, fin, u_row,
      p["w0"], p["b0"].reshape(1, HID),
      p["w1"], p["b1"].reshape(1, HID),
      p["w2"], p["b2"].reshape(1, HID))


_BIG = 3.4e38
_N_SLAB = C_SEL // K


def _roll(x, d):
    return jnp.concatenate([x[:, d:], x[:, :d]], axis=1)


def _ce_stage(vals, idx, j, desc):
    w = vals.shape[1]
    down_v, up_v = _roll(vals, j), _roll(vals, w - j)
    down_i, up_i = _roll(idx, j), _roll(idx, w - j)
    lane = lax.broadcasted_iota(jnp.int32, vals.shape, 1)
    is_hi = (lane & j) != 0
    pv = jnp.where(is_hi, up_v, down_v)
    pi = jnp.where(is_hi, up_i, down_i)
    want_max = jnp.logical_xor(desc, is_hi)
    keep = (want_max & (vals >= pv)) | (~want_max & (vals <= pv))
    return jnp.where(keep, vals, pv), jnp.where(keep, idx, pi)


def _sort_desc(vals, idx):
    lane = lax.broadcasted_iota(jnp.int32, vals.shape, 1)
    for k in (2, 4, 8, 16, 32, 64, 128):
        desc = (lane & k) == 0
        j = k // 2
        while j >= 1:
            vals, idx = _ce_stage(vals, idx, j, desc)
            j //= 2
    return vals, idx


def _slab_minima(d2_ref):
    m = d2_ref[:, 0:K]
    mg = jnp.zeros((A, K), jnp.int32)
    for g in range(1, _N_SLAB):
        s = d2_ref[:, g * K:(g + 1) * K]
        better = s < m
        m = jnp.where(better, s, m)
        mg = jnp.where(better, g, mg)
    return m, mg


def _select_body(xt_ref, a_ref, u_ref, bd_ref, bi_ref, adu_ref, adx_ref,
                 d2_ref, go_ref):
    step = pl.program_id(0)

    @pl.when(step == 0)
    def _():
        bd_ref[...] = jnp.full((A, K), _BIG, jnp.float32)
        bi_ref[...] = jnp.zeros((A, K), jnp.int32)
        aq = _q(a_ref[...]) * _q(u_ref[...])
        adu_bf = aq[:, 0:1] + aq[:, 1:2] + aq[:, 2:3]
        ae = a_ref[...] * u_ref[...]
        adu_ex = ae[:, 0:1] + ae[:, 1:2] + ae[:, 2:3]
        adu_ref[...] = jnp.broadcast_to(adu_bf, (A, K))
        adx_ref[...] = jnp.broadcast_to(adu_ex, (A, K))

    x0 = xt_ref[0:1, :]
    x1 = xt_ref[1:2, :]
    x2 = xt_ref[2:3, :]
    a0 = a_ref[:, 0:1]
    a1 = a_ref[:, 1:2]
    a2 = a_ref[:, 2:3]
    nx2 = x0 * x0 + x1 * x1 + x2 * x2
    na2 = a0 * a0 + a1 * a1 + a2 * a2
    dot = _q(a0) * _q(x0) + _q(a1) * _q(x1) + _q(a2) * _q(x2)
    d2_ref[...] = jnp.maximum(na2 + nx2 - 2.0 * dot, 0.0)

    lane = lax.broadcasted_iota(jnp.int32, (A, K), 1)
    base = step * C_SEL

    m0, _ = _slab_minima(d2_ref)
    go_ref[0] = jnp.max(jnp.where(m0 < bd_ref[:, K - 1:K], 1, 0))

    def _iter(_, carry):
        @pl.when(go_ref[0] != 0)
        def _():
            m, mg = _slab_minima(d2_ref)
            gidx = base + mg * K + lane
            cv, ci = _sort_desc(m, gidx)
            sv, si = bd_ref[...], bi_ref[...]
            take_s = sv <= cv
            mv = jnp.where(take_s, sv, cv)
            mi = jnp.where(take_s, si, ci)
            for j in (64, 32, 16, 8, 4, 2, 1):
                mv, mi = _ce_stage(mv, mi, j, False)
            bd_ref[...] = mv
            bi_ref[...] = mi
            for g in range(_N_SLAB):
                s = d2_ref[:, g * K:(g + 1) * K]
                d2_ref[:, g * K:(g + 1) * K] = jnp.where(mg == g, _BIG, s)
            m2, _ = _slab_minima(d2_ref)
            go_ref[0] = jnp.max(jnp.where(m2 < mv[:, K - 1:K], 1, 0))
        return carry

    lax.fori_loop(0, _N_SLAB, _iter, 0)


def _run_select(xt, a, u_row):
    grid = N_PAD // C_SEL
    return pl.pallas_call(
        _select_body,
        grid=(grid,),
        in_specs=[
            pl.BlockSpec((3, C_SEL), lambda i: (0, i)),
            pl.BlockSpec((A, 3), lambda i: (0, 0)),
            pl.BlockSpec((1, 3), lambda i: (0, 0)),
        ],
        out_specs=[
            pl.BlockSpec((A, K), lambda i: (0, 0)),
            pl.BlockSpec((A, K), lambda i: (0, 0)),
            pl.BlockSpec((A, K), lambda i: (0, 0)),
            pl.BlockSpec((A, K), lambda i: (0, 0)),
        ],
        out_shape=[
            jax.ShapeDtypeStruct((A, K), jnp.float32),
            jax.ShapeDtypeStruct((A, K), jnp.int32),
            jax.ShapeDtypeStruct((A, K), jnp.float32),
            jax.ShapeDtypeStruct((A, K), jnp.float32),
        ],
        scratch_shapes=[
            pltpu.VMEM((A, C_SEL), jnp.float32),
            pltpu.SMEM((1,), jnp.int32),
        ],
    )(xt, a, u_row)


_NC, _NS = 2, 16
_NW = _NC * _NS
_B_TOT = A * K
_B_PER_W = _B_TOT // _NW
_B_CHUNK = 128


def _gather_rows(table, idx_flat):
    mesh = plsc.VectorSubcoreMesh(core_axis_name="c", subcore_axis_name="s")

    @functools.partial(
        pl.kernel,
        out_type=jax.ShapeDtypeStruct((_B_TOT, TBL_W), jnp.float32),
        mesh=mesh,
        scratch_types=[
            pltpu.VMEM((_B_CHUNK,), jnp.int32),
            pltpu.VMEM((_B_CHUNK, TBL_W), jnp.float32),
            pltpu.SemaphoreType.DMA,
        ],
    )
    def gk(table_hbm, idx_hbm, out_hbm, idx_v, rows_v, sem):
        wid = lax.axis_index("s") * _NC + lax.axis_index("c")
        for j in range(_B_PER_W // _B_CHUNK):
            base = wid * _B_PER_W + j * _B_CHUNK
            pltpu.sync_copy(idx_hbm.at[pl.ds(base, _B_CHUNK)], idx_v)
            pltpu.async_copy(table_hbm.at[idx_v], rows_v, sem).wait()
            pltpu.sync_copy(rows_v, out_hbm.at[pl.ds(base, _B_CHUNK)])

    return gk(table, idx_flat)


def _edge_body(g_ref, d2k_ref, adu_ref, adx_ref,
               w0f, w0d2, w0adu, w0ndu, w0rdu, b0, w1, b1, w2, b2,
               gw0, gb0, gw1, gb1,
               eh_ref, lg_ref):
    g = g_ref[...]
    feat = g[:, :HID]
    ndu = g[:, HID:HID + 1]
    ndu_ex = g[:, HID + 1:HID + 2]
    d2k = d2k_ref[...]
    adu = adu_ref[...]
    rdu = adx_ref[...] - ndu_ex
    h = _dot_bf(feat, w0f[...])
    h = h + _q(d2k) * _q(w0d2[...]) + _q(adu) * _q(w0adu[...]) \
        + _q(ndu) * _q(w0ndu[...]) + _q(rdu) * _q(w0rdu[...]) + b0[...]
    h = _gelu(h)
    h = _gelu(_dot_bf(h, w1[...]) + b1[...])
    eh = _dot_bf(h, w2[...]) + b2[...]
    eh_ref[...] = eh
    hg = _gelu(_dot_bf(eh, gw0[...]) + gb0[...])
    lg = _dot_bf(hg, gw1[...]) + gb1[...]
    lg_ref[...] = lg - d2k


def _run_edge(g, d2k_col, adu_col, adx_col, pe, pg):
    grid = _B_TOT // (AB * K)
    rows = AB * K
    full = lambda shape: pl.BlockSpec(shape, lambda i: (0, 0))
    w0 = pe["w0"]
    return pl.pallas_call(
        _edge_body,
        grid=(grid,),
        in_specs=[
            pl.BlockSpec((rows, TBL_W), lambda i: (i, 0)),
            pl.BlockSpec((rows, 1), lambda i: (i, 0)),
            pl.BlockSpec((rows, 1), lambda i: (i, 0)),
            pl.BlockSpec((rows, 1), lambda i: (i, 0)),
            full((HID, HID)), full((1, HID)), full((1, HID)),
            full((1, HID)), full((1, HID)), full((1, HID)),
            full((HID, HID)), full((1, HID)),
            full((HID, HID)), full((1, HID)),
            full((HID, HID)), full((1, HID)),
            full((HID, 1)), full((1, 1)),
        ],
        out_specs=[
            pl.BlockSpec((rows, HID), lambda i: (i, 0)),
            pl.BlockSpec((rows, 1), lambda i: (i, 0)),
        ],
        out_shape=[
            jax.ShapeDtypeStruct((_B_TOT, HID), jnp.float32),
            jax.ShapeDtypeStruct((_B_TOT, 1), jnp.float32),
        ],
    )(g, d2k_col, adu_col, adx_col,
      w0[:HID, :], w0[HID:HID + 1, :], w0[HID + 1:HID + 2, :],
      w0[HID + 2:HID + 3, :], w0[HID + 3:HID + 4, :],
      pe["b0"].reshape(1, HID),
      pe["w1"], pe["b1"].reshape(1, HID),
      pe["w2"], pe["b2"].reshape(1, HID),
      pg["w0"], pg["b0"].reshape(1, HID),
      pg["w1"], pg["b1"].reshape(1, 1))


def _softmax_body(l_ref, w_ref):
    l = l_ref[...]
    m = jnp.max(l, axis=1, keepdims=True)
    e = jnp.exp(l - m)
    w_ref[...] = e / jnp.sum(e, axis=1, keepdims=True)


def _run_softmax(lg):
    return pl.pallas_call(
        _softmax_body,
        in_specs=[pl.BlockSpec((A, K), lambda: (0, 0))],
        out_specs=pl.BlockSpec((A, K), lambda: (0, 0)),
        out_shape=jax.ShapeDtypeStruct((A, K), jnp.float32),
    )(lg)


def _agg_body(eh_ref, wc_ref, fsum_ref,
              gw0, gb0, gw1, gb1, gw2, gb2,
              ow0, ob0, ow1, ob1, ow2, ob2,
              out_ref):
    weh = eh_ref[...] * wc_ref[...]
    rows = weh.shape[0]
    rblk = lax.broadcasted_iota(jnp.int32, (AB, rows), 1) // K
    cblk = lax.broadcasted_iota(jnp.int32, (AB, rows), 0)
    sel = (rblk == cblk).astype(jnp.float32)
    agg = jnp.dot(sel, weh, preferred_element_type=jnp.float32, precision=lax.Precision.HIGHEST)
    mean = fsum_ref[...] * (1.0 / N)
    gc = _mlp3(mean, gw0[...], gb0[...], gw1[...], gb1[...],
               gw2[...], gb2[...])
    gb = jnp.broadcast_to(gc, (AB, HID))
    oi = jnp.concatenate([agg, gb], axis=1)
    out_ref[...] = _mlp3(oi, ow0[...], ob0[...], ow1[...], ob1[...],
                         ow2[...], ob2[...])


def _run_agg(eh, w_col, fsum, pglob, pout):
    grid = A // AB
    rows = AB * K
    full = lambda shape: pl.BlockSpec(shape, lambda i: (0, 0))
    return pl.pallas_call(
        _agg_body,
        grid=(grid,),
        in_specs=[
            pl.BlockSpec((rows, HID), lambda i: (i, 0)),
            pl.BlockSpec((rows, 1), lambda i: (i, 0)),
            full((1, HID)),
            full((HID, HID)), full((1, HID)),
            full((HID, HID)), full((1, HID)),
            full((HID, HID)), full((1, HID)),
            full((2 * HID, HID)), full((1, HID)),
            full((HID, HID)), full((1, HID)),
            full((HID, HID)), full((1, HID)),
        ],
        out_specs=pl.BlockSpec((AB, HID), lambda i: (i, 0)),
        out_shape=jax.ShapeDtypeStruct((A, HID), jnp.float32),
    )(eh, w_col, fsum,
      pglob["w0"], pglob["b0"].reshape(1, HID),
      pglob["w1"], pglob["b1"].reshape(1, HID),
      pglob["w2"], pglob["b2"].reshape(1, HID),
      pout["w0"], pout["b0"].reshape(1, HID),
      pout["w1"], pout["b1"].reshape(1, HID),
      pout["w2"], pout["b2"].reshape(1, HID))


def kernel(input_coords, input_x, anchor_coords, flow_dir, params):
    x = input_coords[0]
    fin = input_x[0]
    a = anchor_coords[0]
    u = flow_dir[0]
    u = u / (jnp.linalg.norm(u) + 1e-08)
    u_row = u.reshape(1, 3)

    table, fsum = _run_prep(x, fin, u_row, params["ne"])
    xt = jnp.pad(x.T, ((0, 0), (0, N_PAD - N)), constant_values=1e15)
    best_d, best_i, adu, adx = _run_select(xt, a, u_row)

    idx_flat = best_i.reshape(_B_TOT)
    g = _gather_rows(table, idx_flat)

    d2k_col = best_d.reshape(_B_TOT, 1)
    adu_col = adu.reshape(_B_TOT, 1)
    adx_col = adx.reshape(_B_TOT, 1)
    eh, lg = _run_edge(g, d2k_col, adu_col, adx_col,
                       params["edge"], params["gate"])

    w = _run_softmax(lg.reshape(A, K))
    w_col = w.reshape(_B_TOT, 1)
    anchor_feat = _run_agg(eh, w_col, fsum, params["glob"], params["out"])
    return anchor_feat[None]


# speedup vs baseline: 9.2709x; 1.2131x over previous
import functools

import jax
import jax.numpy as jnp
from jax import lax
from jax.experimental import pallas as pl
from jax.experimental.pallas import tpu as pltpu
from jax.experimental.pallas import tpu_sc as plsc

N = 100000
A = 512
K = 128
IN_CH = 16
HID = 64
TBL_W = 128
C_SEL = 4096
N_PAD = 102400
N_BLK = 2000
AB = 32

_SQRT_HALF = 0.7071067811865476


def _gelu(x):
    return 0.5 * x * (1.0 + lax.erf(x * _SQRT_HALF))


def _q(x):
    return x.astype(jnp.bfloat16).astype(jnp.float32)


def _dot_bf(x, w):
    return jnp.dot(x.astype(jnp.bfloat16), w.astype(jnp.bfloat16),
                   preferred_element_type=jnp.float32)


def _mlp3(x, w0, b0, w1, b1, w2, b2):
    h = _gelu(_dot_bf(x, w0) + b0)
    h = _gelu(_dot_bf(h, w1) + b1)
    return _dot_bf(h, w2) + b2


def _prep_body(x_ref, fin_ref, u_ref, w0, b0, w1, b1, w2, b2,
               tbl_ref, fsum_ref):
    feat = _mlp3(fin_ref[...], w0[...], b0[...], w1[...], b1[...],
                 w2[...], b2[...])
    xq = _q(x_ref[...])
    uq = _q(u_ref[...])
    pq = xq * uq
    ndu_bf = pq[:, 0:1] + pq[:, 1:2] + pq[:, 2:3]
    pe = x_ref[...] * u_ref[...]
    ndu_ex = pe[:, 0:1] + pe[:, 1:2] + pe[:, 2:3]
    pad = jnp.zeros((feat.shape[0], TBL_W - HID - 2), jnp.float32)
    tbl_ref[...] = jnp.concatenate([feat, ndu_bf, ndu_ex, pad], axis=1)
    part = jnp.sum(feat, axis=0, keepdims=True)

    @pl.when(pl.program_id(0) == 0)
    def _():
        fsum_ref[...] = part

    @pl.when(pl.program_id(0) != 0)
    def _():
        fsum_ref[...] += part


def _run_prep(x, fin, u_row, p):
    grid = N // N_BLK
    full = lambda shape: pl.BlockSpec(shape, lambda i: (0, 0))
    return pl.pallas_call(
        _prep_body,
        grid=(grid,),
        in_specs=[
            pl.BlockSpec((N_BLK, 3), lambda i: (i, 0)),
            pl.BlockSpec((N_BLK, IN_CH), lambda i: (i, 0)),
            full((1, 3)),
            full((IN_CH, HID)), full((1, HID)),
            full((HID, HID)), full((1, HID)),
            full((HID, HID)), full((1, HID)),
        ],
        out_specs=[
            pl.BlockSpec((N_BLK, TBL_W), lambda i: (i, 0)),
            pl.BlockSpec((1, HID), lambda i: (0, 0)),
        ],
        out_shape=[
            jax.ShapeDtypeStruct((N, TBL_W), jnp.float32),
            jax.ShapeDtypeStruct((1, HID), jnp.float32),
        ],
    )(x, fin, u_row,
      p["w0"], p["b0"].reshape(1, HID),
      p["w1"], p["b1"].reshape(1, HID),
      p["w2"], p["b2"].reshape(1, HID))


_BIG = 3.4e38
_N_SLAB = C_SEL // K


def _roll(x, d):
    return jnp.concatenate([x[:, d:], x[:, :d]], axis=1)


def _ce_stage(vals, idx, j, desc):
    w = vals.shape[1]
    down_v, up_v = _roll(vals, j), _roll(vals, w - j)
    down_i, up_i = _roll(idx, j), _roll(idx, w - j)
    lane = lax.broadcasted_iota(jnp.int32, vals.shape, 1)
    is_hi = (lane & j) != 0
    pv = jnp.where(is_hi, up_v, down_v)
    pi = jnp.where(is_hi, up_i, down_i)
    want_max = jnp.logical_xor(desc, is_hi)
    keep = (want_max & (vals >= pv)) | (~want_max & (vals <= pv))
    return jnp.where(keep, vals, pv), jnp.where(keep, idx, pi)


def _sort_desc(vals, idx):
    lane = lax.broadcasted_iota(jnp.int32, vals.shape, 1)
    for k in (2, 4, 8, 16, 32, 64, 128):
        desc = (lane & k) == 0
        j = k // 2
        while j >= 1:
            vals, idx = _ce_stage(vals, idx, j, desc)
            j //= 2
    return vals, idx


def _min_upd(m, mg, s, g):
    better = s < m
    return jnp.where(better, s, m), jnp.where(better, g, mg)


def _select_body(xt_ref, a_ref, u_ref, bd_ref, bi_ref, adu_ref, adx_ref,
                 d2_ref):
    step = pl.program_id(0)

    @pl.when(step == 0)
    def _():
        bd_ref[...] = jnp.full((A, K), _BIG, jnp.float32)
        bi_ref[...] = jnp.zeros((A, K), jnp.int32)
        aq = _q(a_ref[...]) * _q(u_ref[...])
        adu_bf = aq[:, 0:1] + aq[:, 1:2] + aq[:, 2:3]
        ae = a_ref[...] * u_ref[...]
        adu_ex = ae[:, 0:1] + ae[:, 1:2] + ae[:, 2:3]
        adu_ref[...] = jnp.broadcast_to(adu_bf, (A, K))
        adx_ref[...] = jnp.broadcast_to(adu_ex, (A, K))

    a0 = a_ref[:, 0:1]
    a1 = a_ref[:, 1:2]
    a2 = a_ref[:, 2:3]
    na2 = a0 * a0 + a1 * a1 + a2 * a2
    a0q, a1q, a2q = _q(a0), _q(a1), _q(a2)

    m = jnp.full((A, K), _BIG, jnp.float32)
    mg = jnp.zeros((A, K), jnp.int32)
    for g in range(_N_SLAB):
        sl = slice(g * K, (g + 1) * K)
        x0 = xt_ref[0:1, sl]
        x1 = xt_ref[1:2, sl]
        x2 = xt_ref[2:3, sl]
        nx2 = x0 * x0 + x1 * x1 + x2 * x2
        dot = a0q * _q(x0) + a1q * _q(x1) + a2q * _q(x2)
        s = jnp.maximum(na2 + nx2 - 2.0 * dot, 0.0)
        d2_ref[:, sl] = s
        m, mg = _min_upd(m, mg, s, g)

    lane = lax.broadcasted_iota(jnp.int32, (A, K), 1)
    base = step * C_SEL
    go0 = jnp.max(jnp.where(m < bd_ref[:, K - 1:K], 1, 0))

    def _iter(_, carry):
        go, m, mg = carry

        def _run(m, mg):
            cv, ci = _sort_desc(m, base + mg * K + lane)
            sv, si = bd_ref[...], bi_ref[...]
            take_s = sv <= cv
            mv = jnp.where(take_s, sv, cv)
            mi = jnp.where(take_s, si, ci)
            for j in (64, 32, 16, 8, 4, 2, 1):
                mv, mi = _ce_stage(mv, mi, j, False)
            bd_ref[...] = mv
            bi_ref[...] = mi
            m2 = jnp.full((A, K), _BIG, jnp.float32)
            mg2 = jnp.zeros((A, K), jnp.int32)
            for g in range(_N_SLAB):
                sl = slice(g * K, (g + 1) * K)
                s = jnp.where(mg == g, _BIG, d2_ref[:, sl])
                d2_ref[:, sl] = s
                m2, mg2 = _min_upd(m2, mg2, s, g)
            go2 = jnp.max(jnp.where(m2 < mv[:, K - 1:K], 1, 0))
            return go2, m2, mg2

        def _skip(m, mg):
            return jnp.int32(0), m, mg

        return lax.cond(go != 0, _run, _skip, m, mg)

    lax.fori_loop(0, _N_SLAB, _iter, (go0, m, mg))


def _run_select(xt, a, u_row):
    grid = N_PAD // C_SEL
    return pl.pallas_call(
        _select_body,
        grid=(grid,),
        in_specs=[
            pl.BlockSpec((3, C_SEL), lambda i: (0, i)),
            pl.BlockSpec((A, 3), lambda i: (0, 0)),
            pl.BlockSpec((1, 3), lambda i: (0, 0)),
        ],
        out_specs=[
            pl.BlockSpec((A, K), lambda i: (0, 0)),
            pl.BlockSpec((A, K), lambda i: (0, 0)),
            pl.BlockSpec((A, K), lambda i: (0, 0)),
            pl.BlockSpec((A, K), lambda i: (0, 0)),
        ],
        out_shape=[
            jax.ShapeDtypeStruct((A, K), jnp.float32),
            jax.ShapeDtypeStruct((A, K), jnp.int32),
            jax.ShapeDtypeStruct((A, K), jnp.float32),
            jax.ShapeDtypeStruct((A, K), jnp.float32),
        ],
        scratch_shapes=[
            pltpu.VMEM((A, C_SEL), jnp.float32),
        ],
    )(xt, a, u_row)


_NC, _NS = 2, 16
_NW = _NC * _NS
_B_TOT = A * K
_B_PER_W = _B_TOT // _NW
_B_CHUNK = 128


def _gather_rows(table, idx_flat):
    mesh = plsc.VectorSubcoreMesh(core_axis_name="c", subcore_axis_name="s")

    @functools.partial(
        pl.kernel,
        out_type=jax.ShapeDtypeStruct((_B_TOT, TBL_W), jnp.float32),
        mesh=mesh,
        scratch_types=[
            pltpu.VMEM((_B_CHUNK,), jnp.int32),
            pltpu.VMEM((_B_CHUNK, TBL_W), jnp.float32),
            pltpu.SemaphoreType.DMA,
        ],
    )
    def gk(table_hbm, idx_hbm, out_hbm, idx_v, rows_v, sem):
        wid = lax.axis_index("s") * _NC + lax.axis_index("c")
        for j in range(_B_PER_W // _B_CHUNK):
            base = wid * _B_PER_W + j * _B_CHUNK
            pltpu.sync_copy(idx_hbm.at[pl.ds(base, _B_CHUNK)], idx_v)
            pltpu.async_copy(table_hbm.at[idx_v], rows_v, sem).wait()
            pltpu.sync_copy(rows_v, out_hbm.at[pl.ds(base, _B_CHUNK)])

    return gk(table, idx_flat)


def _edge_body(g_ref, d2k_ref, adu_ref, adx_ref,
               w0f, w0d2, w0adu, w0ndu, w0rdu, b0, w1, b1, w2, b2,
               gw0, gb0, gw1, gb1,
               eh_ref, lg_ref):
    g = g_ref[...]
    feat = g[:, :HID]
    ndu = g[:, HID:HID + 1]
    ndu_ex = g[:, HID + 1:HID + 2]
    d2k = d2k_ref[...]
    adu = adu_ref[...]
    rdu = adx_ref[...] - ndu_ex
    h = _dot_bf(feat, w0f[...])
    h = h + _q(d2k) * _q(w0d2[...]) + _q(adu) * _q(w0adu[...]) \
        + _q(ndu) * _q(w0ndu[...]) + _q(rdu) * _q(w0rdu[...]) + b0[...]
    h = _gelu(h)
    h = _gelu(_dot_bf(h, w1[...]) + b1[...])
    eh = _dot_bf(h, w2[...]) + b2[...]
    eh_ref[...] = eh
    hg = _gelu(_dot_bf(eh, gw0[...]) + gb0[...])
    lg = _dot_bf(hg, gw1[...]) + gb1[...]
    lg_ref[...] = lg - d2k


def _run_edge(g, d2k_col, adu_col, adx_col, pe, pg):
    grid = _B_TOT // (AB * K)
    rows = AB * K
    full = lambda shape: pl.BlockSpec(shape, lambda i: (0, 0))
    w0 = pe["w0"]
    return pl.pallas_call(
        _edge_body,
        grid=(grid,),
        in_specs=[
            pl.BlockSpec((rows, TBL_W), lambda i: (i, 0)),
            pl.BlockSpec((rows, 1), lambda i: (i, 0)),
            pl.BlockSpec((rows, 1), lambda i: (i, 0)),
            pl.BlockSpec((rows, 1), lambda i: (i, 0)),
            full((HID, HID)), full((1, HID)), full((1, HID)),
            full((1, HID)), full((1, HID)), full((1, HID)),
            full((HID, HID)), full((1, HID)),
            full((HID, HID)), full((1, HID)),
            full((HID, HID)), full((1, HID)),
            full((HID, 1)), full((1, 1)),
        ],
        out_specs=[
            pl.BlockSpec((rows, HID), lambda i: (i, 0)),
            pl.BlockSpec((rows, 1), lambda i: (i, 0)),
        ],
        out_shape=[
            jax.ShapeDtypeStruct((_B_TOT, HID), jnp.float32),
            jax.ShapeDtypeStruct((_B_TOT, 1), jnp.float32),
        ],
    )(g, d2k_col, adu_col, adx_col,
      w0[:HID, :], w0[HID:HID + 1, :], w0[HID + 1:HID + 2, :],
      w0[HID + 2:HID + 3, :], w0[HID + 3:HID + 4, :],
      pe["b0"].reshape(1, HID),
      pe["w1"], pe["b1"].reshape(1, HID),
      pe["w2"], pe["b2"].reshape(1, HID),
      pg["w0"], pg["b0"].reshape(1, HID),
      pg["w1"], pg["b1"].reshape(1, 1))


def _softmax_body(l_ref, w_ref):
    l = l_ref[...]
    m = jnp.max(l, axis=1, keepdims=True)
    e = jnp.exp(l - m)
    w_ref[...] = e / jnp.sum(e, axis=1, keepdims=True)


def _run_softmax(lg):
    return pl.pallas_call(
        _softmax_body,
        in_specs=[pl.BlockSpec((A, K), lambda: (0, 0))],
        out_specs=pl.BlockSpec((A, K), lambda: (0, 0)),
        out_shape=jax.ShapeDtypeStruct((A, K), jnp.float32),
    )(lg)


def _agg_body(eh_ref, wc_ref, fsum_ref,
              gw0, gb0, gw1, gb1, gw2, gb2,
              ow0, ob0, ow1, ob1, ow2, ob2,
              out_ref):
    weh = eh_ref[...] * wc_ref[...]
    rows = weh.shape[0]
    rblk = lax.broadcasted_iota(jnp.int32, (AB, rows), 1) // K
    cblk = lax.broadcasted_iota(jnp.int32, (AB, rows), 0)
    sel = (rblk == cblk).astype(jnp.float32)
    agg = jnp.dot(sel, weh, preferred_element_type=jnp.float32, precision=lax.Precision.HIGHEST)
    mean = fsum_ref[...] * (1.0 / N)
    gc = _mlp3(mean, gw0[...], gb0[...], gw1[...], gb1[...],
               gw2[...], gb2[...])
    gb = jnp.broadcast_to(gc, (AB, HID))
    oi = jnp.concatenate([agg, gb], axis=1)
    out_ref[...] = _mlp3(oi, ow0[...], ob0[...], ow1[...], ob1[...],
                         ow2[...], ob2[...])


def _run_agg(eh, w_col, fsum, pglob, pout):
    grid = A // AB
    rows = AB * K
    full = lambda shape: pl.BlockSpec(shape, lambda i: (0, 0))
    return pl.pallas_call(
        _agg_body,
        grid=(grid,),
        in_specs=[
            pl.BlockSpec((rows, HID), lambda i: (i, 0)),
            pl.BlockSpec((rows, 1), lambda i: (i, 0)),
            full((1, HID)),
            full((HID, HID)), full((1, HID)),
            full((HID, HID)), full((1, HID)),
            full((HID, HID)), full((1, HID)),
            full((2 * HID, HID)), full((1, HID)),
            full((HID, HID)), full((1, HID)),
            full((HID, HID)), full((1, HID)),
        ],
        out_specs=pl.BlockSpec((AB, HID), lambda i: (i, 0)),
        out_shape=jax.ShapeDtypeStruct((A, HID), jnp.float32),
    )(eh, w_col, fsum,
      pglob["w0"], pglob["b0"].reshape(1, HID),
      pglob["w1"], pglob["b1"].reshape(1, HID),
      pglob["w2"], pglob["b2"].reshape(1, HID),
      pout["w0"], pout["b0"].reshape(1, HID),
      pout["w1"], pout["b1"].reshape(1, HID),
      pout["w2"], pout["b2"].reshape(1, HID))


def kernel(input_coords, input_x, anchor_coords, flow_dir, params):
    x = input_coords[0]
    fin = input_x[0]
    a = anchor_coords[0]
    u = flow_dir[0]
    u = u / (jnp.linalg.norm(u) + 1e-08)
    u_row = u.reshape(1, 3)

    table, fsum = _run_prep(x, fin, u_row, params["ne"])
    xt = jnp.pad(x.T, ((0, 0), (0, N_PAD - N)), constant_values=1e15)
    best_d, best_i, adu, adx = _run_select(xt, a, u_row)

    idx_flat = best_i.reshape(_B_TOT)
    g = _gather_rows(table, idx_flat)

    d2k_col = best_d.reshape(_B_TOT, 1)
    adu_col = adu.reshape(_B_TOT, 1)
    adx_col = adx.reshape(_B_TOT, 1)
    eh, lg = _run_edge(g, d2k_col, adu_col, adx_col,
                       params["edge"], params["gate"])

    w = _run_softmax(lg.reshape(A, K))
    w_col = w.reshape(_B_TOT, 1)
    anchor_feat = _run_agg(eh, w_col, fsum, params["glob"], params["out"])
    return anchor_feat[None]


# speedup vs baseline: 10.0197x; 1.0808x over previous
import functools

import jax
import jax.numpy as jnp
from jax import lax
from jax.experimental import pallas as pl
from jax.experimental.pallas import tpu as pltpu
from jax.experimental.pallas import tpu_sc as plsc

N = 100000
A = 512
K = 128
IN_CH = 16
HID = 64
TBL_W = 128
C_SEL = 4096
N_PAD = 102400
N_BLK = 2000
AB = 32

_SQRT_HALF = 0.7071067811865476


def _gelu(x):
    return 0.5 * x * (1.0 + lax.erf(x * _SQRT_HALF))


def _q(x):
    return x.astype(jnp.bfloat16).astype(jnp.float32)


def _dot_bf(x, w):
    return jnp.dot(x.astype(jnp.bfloat16), w.astype(jnp.bfloat16),
                   preferred_element_type=jnp.float32)


def _mlp3(x, w0, b0, w1, b1, w2, b2):
    h = _gelu(_dot_bf(x, w0) + b0)
    h = _gelu(_dot_bf(h, w1) + b1)
    return _dot_bf(h, w2) + b2


def _prep_body(x_ref, fin_ref, u_ref, w0, b0, w1, b1, w2, b2,
               tbl_ref, fsum_ref):
    feat = _mlp3(fin_ref[...], w0[...], b0[...], w1[...], b1[...],
                 w2[...], b2[...])
    xq = _q(x_ref[...])
    uq = _q(u_ref[...])
    pq = xq * uq
    ndu_bf = pq[:, 0:1] + pq[:, 1:2] + pq[:, 2:3]
    pe = x_ref[...] * u_ref[...]
    ndu_ex = pe[:, 0:1] + pe[:, 1:2] + pe[:, 2:3]
    pad = jnp.zeros((feat.shape[0], TBL_W - HID - 2), jnp.float32)
    tbl_ref[...] = jnp.concatenate([feat, ndu_bf, ndu_ex, pad], axis=1)
    part = jnp.sum(feat, axis=0, keepdims=True)

    @pl.when(pl.program_id(0) == 0)
    def _():
        fsum_ref[...] = part

    @pl.when(pl.program_id(0) != 0)
    def _():
        fsum_ref[...] += part


def _run_prep(x, fin, u_row, p):
    grid = N // N_BLK
    full = lambda shape: pl.BlockSpec(shape, lambda i: (0, 0))
    return pl.pallas_call(
        _prep_body,
        grid=(grid,),
        in_specs=[
            pl.BlockSpec((N_BLK, 3), lambda i: (i, 0)),
            pl.BlockSpec((N_BLK, IN_CH), lambda i: (i, 0)),
            full((1, 3)),
            full((IN_CH, HID)), full((1, HID)),
            full((HID, HID)), full((1, HID)),
            full((HID, HID)), full((1, HID)),
        ],
        out_specs=[
            pl.BlockSpec((N_BLK, TBL_W), lambda i: (i, 0)),
            pl.BlockSpec((1, HID), lambda i: (0, 0)),
        ],
        out_shape=[
            jax.ShapeDtypeStruct((N, TBL_W), jnp.float32),
            jax.ShapeDtypeStruct((1, HID), jnp.float32),
        ],
    )(x, fin, u_row,
      p["w0"], p["b0"].reshape(1, HID),
      p["w1"], p["b1"].reshape(1, HID),
      p["w2"], p["b2"].reshape(1, HID))


_BIG = 3.4e38
_N_SLAB = C_SEL // K


def _roll(x, d):
    return jnp.concatenate([x[:, d:], x[:, :d]], axis=1)


def _ce_stage(vals, idx, j, desc):
    w = vals.shape[1]
    down_v, up_v = _roll(vals, j), _roll(vals, w - j)
    down_i, up_i = _roll(idx, j), _roll(idx, w - j)
    lane = lax.broadcasted_iota(jnp.int32, vals.shape, 1)
    is_hi = (lane & j) != 0
    pv = jnp.where(is_hi, up_v, down_v)
    pi = jnp.where(is_hi, up_i, down_i)
    want_max = jnp.logical_xor(desc, is_hi)
    keep = (want_max & (vals >= pv)) | (~want_max & (vals <= pv))
    return jnp.where(keep, vals, pv), jnp.where(keep, idx, pi)


def _sort_desc(vals, idx):
    lane = lax.broadcasted_iota(jnp.int32, vals.shape, 1)
    for k in (2, 4, 8, 16, 32, 64, 128):
        desc = (lane & k) == 0
        j = k // 2
        while j >= 1:
            vals, idx = _ce_stage(vals, idx, j, desc)
            j //= 2
    return vals, idx


def _min_upd(m, mg, s, g):
    better = s < m
    return jnp.where(better, s, m), jnp.where(better, g, mg)


def _select_body(xt_ref, a_ref, u_ref, bd_ref, bi_ref, adu_ref, adx_ref,
                 d2_ref):
    step = pl.program_id(0)

    @pl.when(step == 0)
    def _():
        bd_ref[...] = jnp.full((A, K), _BIG, jnp.float32)
        bi_ref[...] = jnp.zeros((A, K), jnp.int32)
        aq = _q(a_ref[...]) * _q(u_ref[...])
        adu_bf = aq[:, 0:1] + aq[:, 1:2] + aq[:, 2:3]
        ae = a_ref[...] * u_ref[...]
        adu_ex = ae[:, 0:1] + ae[:, 1:2] + ae[:, 2:3]
        adu_ref[...] = jnp.broadcast_to(adu_bf, (A, K))
        adx_ref[...] = jnp.broadcast_to(adu_ex, (A, K))

    a0 = a_ref[:, 0:1]
    a1 = a_ref[:, 1:2]
    a2 = a_ref[:, 2:3]
    na2 = a0 * a0 + a1 * a1 + a2 * a2
    a0q, a1q, a2q = _q(a0), _q(a1), _q(a2)

    m = jnp.full((A, K), _BIG, jnp.float32)
    mg = jnp.zeros((A, K), jnp.int32)
    for g in range(_N_SLAB):
        sl = slice(g * K, (g + 1) * K)
        x0 = xt_ref[0:1, sl]
        x1 = xt_ref[1:2, sl]
        x2 = xt_ref[2:3, sl]
        nx2 = x0 * x0 + x1 * x1 + x2 * x2
        dot = a0q * _q(x0) + a1q * _q(x1) + a2q * _q(x2)
        s = jnp.maximum(na2 + nx2 - 2.0 * dot, 0.0)
        d2_ref[:, sl] = s
        m, mg = _min_upd(m, mg, s, g)

    lane = lax.broadcasted_iota(jnp.int32, (A, K), 1)
    base = step * C_SEL
    go0 = jnp.max(jnp.where(m < bd_ref[:, K - 1:K], 1, 0))

    def _cond(carry):
        return carry[0] != 0

    def _body(carry):
        _, m, mg = carry
        cv, ci = _sort_desc(m, base + mg * K + lane)
        sv, si = bd_ref[...], bi_ref[...]
        take_s = sv <= cv
        mv = jnp.where(take_s, sv, cv)
        mi = jnp.where(take_s, si, ci)
        for j in (64, 32, 16, 8, 4, 2, 1):
            mv, mi = _ce_stage(mv, mi, j, False)
        bd_ref[...] = mv
        bi_ref[...] = mi
        m2 = jnp.full((A, K), _BIG, jnp.float32)
        mg2 = jnp.zeros((A, K), jnp.int32)
        for g in range(_N_SLAB):
            sl = slice(g * K, (g + 1) * K)
            s = jnp.where(mg == g, _BIG, d2_ref[:, sl])
            d2_ref[:, sl] = s
            m2, mg2 = _min_upd(m2, mg2, s, g)
        go2 = jnp.max(jnp.where(m2 < mv[:, K - 1:K], 1, 0))
        return go2, m2, mg2

    lax.while_loop(_cond, _body, (go0, m, mg))


def _run_select(xt, a, u_row):
    grid = N_PAD // C_SEL
    return pl.pallas_call(
        _select_body,
        grid=(grid,),
        in_specs=[
            pl.BlockSpec((3, C_SEL), lambda i: (0, i)),
            pl.BlockSpec((A, 3), lambda i: (0, 0)),
            pl.BlockSpec((1, 3), lambda i: (0, 0)),
        ],
        out_specs=[
            pl.BlockSpec((A, K), lambda i: (0, 0)),
            pl.BlockSpec((A, K), lambda i: (0, 0)),
            pl.BlockSpec((A, K), lambda i: (0, 0)),
            pl.BlockSpec((A, K), lambda i: (0, 0)),
        ],
        out_shape=[
            jax.ShapeDtypeStruct((A, K), jnp.float32),
            jax.ShapeDtypeStruct((A, K), jnp.int32),
            jax.ShapeDtypeStruct((A, K), jnp.float32),
            jax.ShapeDtypeStruct((A, K), jnp.float32),
        ],
        scratch_shapes=[
            pltpu.VMEM((A, C_SEL), jnp.float32),
        ],
    )(xt, a, u_row)


_NC, _NS = 2, 16
_NW = _NC * _NS
_B_TOT = A * K
_B_PER_W = _B_TOT // _NW
_B_CHUNK = 128


def _gather_rows(table, idx_flat):
    mesh = plsc.VectorSubcoreMesh(core_axis_name="c", subcore_axis_name="s")

    @functools.partial(
        pl.kernel,
        out_type=jax.ShapeDtypeStruct((_B_TOT, TBL_W), jnp.float32),
        mesh=mesh,
        scratch_types=[
            pltpu.VMEM((_B_CHUNK,), jnp.int32),
            pltpu.VMEM((_B_CHUNK, TBL_W), jnp.float32),
            pltpu.SemaphoreType.DMA,
        ],
    )
    def gk(table_hbm, idx_hbm, out_hbm, idx_v, rows_v, sem):
        wid = lax.axis_index("s") * _NC + lax.axis_index("c")
        for j in range(_B_PER_W // _B_CHUNK):
            base = wid * _B_PER_W + j * _B_CHUNK
            pltpu.sync_copy(idx_hbm.at[pl.ds(base, _B_CHUNK)], idx_v)
            pltpu.async_copy(table_hbm.at[idx_v], rows_v, sem).wait()
            pltpu.sync_copy(rows_v, out_hbm.at[pl.ds(base, _B_CHUNK)])

    return gk(table, idx_flat)


def _edge_body(g_ref, d2k_ref, adu_ref, adx_ref,
               w0f, w0d2, w0adu, w0ndu, w0rdu, b0, w1, b1, w2, b2,
               gw0, gb0, gw1, gb1,
               eh_ref, lg_ref):
    g = g_ref[...]
    feat = g[:, :HID]
    ndu = g[:, HID:HID + 1]
    ndu_ex = g[:, HID + 1:HID + 2]
    d2k = d2k_ref[...]
    adu = adu_ref[...]
    rdu = adx_ref[...] - ndu_ex
    h = _dot_bf(feat, w0f[...])
    h = h + _q(d2k) * _q(w0d2[...]) + _q(adu) * _q(w0adu[...]) \
        + _q(ndu) * _q(w0ndu[...]) + _q(rdu) * _q(w0rdu[...]) + b0[...]
    h = _gelu(h)
    h = _gelu(_dot_bf(h, w1[...]) + b1[...])
    eh = _dot_bf(h, w2[...]) + b2[...]
    eh_ref[...] = eh
    hg = _gelu(_dot_bf(eh, gw0[...]) + gb0[...])
    lg = _dot_bf(hg, gw1[...]) + gb1[...]
    lg_ref[...] = lg - d2k


def _run_edge(g, d2k_col, adu_col, adx_col, pe, pg):
    grid = _B_TOT // (AB * K)
    rows = AB * K
    full = lambda shape: pl.BlockSpec(shape, lambda i: (0, 0))
    w0 = pe["w0"]
    return pl.pallas_call(
        _edge_body,
        grid=(grid,),
        in_specs=[
            pl.BlockSpec((rows, TBL_W), lambda i: (i, 0)),
            pl.BlockSpec((rows, 1), lambda i: (i, 0)),
            pl.BlockSpec((rows, 1), lambda i: (i, 0)),
            pl.BlockSpec((rows, 1), lambda i: (i, 0)),
            full((HID, HID)), full((1, HID)), full((1, HID)),
            full((1, HID)), full((1, HID)), full((1, HID)),
            full((HID, HID)), full((1, HID)),
            full((HID, HID)), full((1, HID)),
            full((HID, HID)), full((1, HID)),
            full((HID, 1)), full((1, 1)),
        ],
        out_specs=[
            pl.BlockSpec((rows, HID), lambda i: (i, 0)),
            pl.BlockSpec((rows, 1), lambda i: (i, 0)),
        ],
        out_shape=[
            jax.ShapeDtypeStruct((_B_TOT, HID), jnp.float32),
            jax.ShapeDtypeStruct((_B_TOT, 1), jnp.float32),
        ],
    )(g, d2k_col, adu_col, adx_col,
      w0[:HID, :], w0[HID:HID + 1, :], w0[HID + 1:HID + 2, :],
      w0[HID + 2:HID + 3, :], w0[HID + 3:HID + 4, :],
      pe["b0"].reshape(1, HID),
      pe["w1"], pe["b1"].reshape(1, HID),
      pe["w2"], pe["b2"].reshape(1, HID),
      pg["w0"], pg["b0"].reshape(1, HID),
      pg["w1"], pg["b1"].reshape(1, 1))


def _softmax_body(l_ref, w_ref):
    l = l_ref[...]
    m = jnp.max(l, axis=1, keepdims=True)
    e = jnp.exp(l - m)
    w_ref[...] = e / jnp.sum(e, axis=1, keepdims=True)


def _run_softmax(lg):
    return pl.pallas_call(
        _softmax_body,
        in_specs=[pl.BlockSpec((A, K), lambda: (0, 0))],
        out_specs=pl.BlockSpec((A, K), lambda: (0, 0)),
        out_shape=jax.ShapeDtypeStruct((A, K), jnp.float32),
    )(lg)


def _agg_body(eh_ref, wc_ref, fsum_ref,
              gw0, gb0, gw1, gb1, gw2, gb2,
              ow0, ob0, ow1, ob1, ow2, ob2,
              out_ref):
    weh = eh_ref[...] * wc_ref[...]
    rows = weh.shape[0]
    rblk = lax.broadcasted_iota(jnp.int32, (AB, rows), 1) // K
    cblk = lax.broadcasted_iota(jnp.int32, (AB, rows), 0)
    sel = (rblk == cblk).astype(jnp.float32)
    agg = jnp.dot(sel, weh, preferred_element_type=jnp.float32, precision=lax.Precision.HIGHEST)
    mean = fsum_ref[...] * (1.0 / N)
    gc = _mlp3(mean, gw0[...], gb0[...], gw1[...], gb1[...],
               gw2[...], gb2[...])
    gb = jnp.broadcast_to(gc, (AB, HID))
    oi = jnp.concatenate([agg, gb], axis=1)
    out_ref[...] = _mlp3(oi, ow0[...], ob0[...], ow1[...], ob1[...],
                         ow2[...], ob2[...])


def _run_agg(eh, w_col, fsum, pglob, pout):
    grid = A // AB
    rows = AB * K
    full = lambda shape: pl.BlockSpec(shape, lambda i: (0, 0))
    return pl.pallas_call(
        _agg_body,
        grid=(grid,),
        in_specs=[
            pl.BlockSpec((rows, HID), lambda i: (i, 0)),
            pl.BlockSpec((rows, 1), lambda i: (i, 0)),
            full((1, HID)),
            full((HID, HID)), full((1, HID)),
            full((HID, HID)), full((1, HID)),
            full((HID, HID)), full((1, HID)),
            full((2 * HID, HID)), full((1, HID)),
            full((HID, HID)), full((1, HID)),
            full((HID, HID)), full((1, HID)),
        ],
        out_specs=pl.BlockSpec((AB, HID), lambda i: (i, 0)),
        out_shape=jax.ShapeDtypeStruct((A, HID), jnp.float32),
    )(eh, w_col, fsum,
      pglob["w0"], pglob["b0"].reshape(1, HID),
      pglob["w1"], pglob["b1"].reshape(1, HID),
      pglob["w2"], pglob["b2"].reshape(1, HID),
      pout["w0"], pout["b0"].reshape(1, HID),
      pout["w1"], pout["b1"].reshape(1, HID),
      pout["w2"], pout["b2"].reshape(1, HID))


def kernel(input_coords, input_x, anchor_coords, flow_dir, params):
    x = input_coords[0]
    fin = input_x[0]
    a = anchor_coords[0]
    u = flow_dir[0]
    u = u / (jnp.linalg.norm(u) + 1e-08)
    u_row = u.reshape(1, 3)

    table, fsum = _run_prep(x, fin, u_row, params["ne"])
    xt = jnp.pad(x.T, ((0, 0), (0, N_PAD - N)), constant_values=1e15)
    best_d, best_i, adu, adx = _run_select(xt, a, u_row)

    idx_flat = best_i.reshape(_B_TOT)
    g = _gather_rows(table, idx_flat)

    d2k_col = best_d.reshape(_B_TOT, 1)
    adu_col = adu.reshape(_B_TOT, 1)
    adx_col = adx.reshape(_B_TOT, 1)
    eh, lg = _run_edge(g, d2k_col, adu_col, adx_col,
                       params["edge"], params["gate"])

    w = _run_softmax(lg.reshape(A, K))
    w_col = w.reshape(_B_TOT, 1)
    anchor_feat = _run_agg(eh, w_col, fsum, params["glob"], params["out"])
    return anchor_feat[None]


# speedup vs baseline: 11.1396x; 1.1118x over previous
import functools

import jax
import jax.numpy as jnp
from jax import lax
from jax.experimental import pallas as pl
from jax.experimental.pallas import tpu as pltpu
from jax.experimental.pallas import tpu_sc as plsc

N = 100000
A = 512
K = 128
IN_CH = 16
HID = 64
TBL_W = 128
C_SEL = 4096
N_PAD = 102400
N_BLK = 2000
AB = 32

_SQRT_HALF = 0.7071067811865476


def _gelu(x):
    return 0.5 * x * (1.0 + lax.erf(x * _SQRT_HALF))


def _q(x):
    return x.astype(jnp.bfloat16).astype(jnp.float32)


def _dot_bf(x, w):
    return jnp.dot(x.astype(jnp.bfloat16), w.astype(jnp.bfloat16),
                   preferred_element_type=jnp.float32)


def _mlp3(x, w0, b0, w1, b1, w2, b2):
    h = _gelu(_dot_bf(x, w0) + b0)
    h = _gelu(_dot_bf(h, w1) + b1)
    return _dot_bf(h, w2) + b2


def _prep_body(x_ref, fin_ref, u_ref, w0, b0, w1, b1, w2, b2,
               tbl_ref, fsum_ref):
    feat = _mlp3(fin_ref[...], w0[...], b0[...], w1[...], b1[...],
                 w2[...], b2[...])
    xq = _q(x_ref[...])
    uq = _q(u_ref[...])
    pq = xq * uq
    ndu_bf = pq[:, 0:1] + pq[:, 1:2] + pq[:, 2:3]
    pe = x_ref[...] * u_ref[...]
    ndu_ex = pe[:, 0:1] + pe[:, 1:2] + pe[:, 2:3]
    pad = jnp.zeros((feat.shape[0], TBL_W - HID - 2), jnp.float32)
    tbl_ref[...] = jnp.concatenate([feat, ndu_bf, ndu_ex, pad], axis=1)
    part = jnp.sum(feat, axis=0, keepdims=True)

    @pl.when(pl.program_id(0) == 0)
    def _():
        fsum_ref[...] = part

    @pl.when(pl.program_id(0) != 0)
    def _():
        fsum_ref[...] += part


def _run_prep(x, fin, u_row, p):
    grid = N // N_BLK
    full = lambda shape: pl.BlockSpec(shape, lambda i: (0, 0))
    return pl.pallas_call(
        _prep_body,
        grid=(grid,),
        in_specs=[
            pl.BlockSpec((N_BLK, 3), lambda i: (i, 0)),
            pl.BlockSpec((N_BLK, IN_CH), lambda i: (i, 0)),
            full((1, 3)),
            full((IN_CH, HID)), full((1, HID)),
            full((HID, HID)), full((1, HID)),
            full((HID, HID)), full((1, HID)),
        ],
        out_specs=[
            pl.BlockSpec((N_BLK, TBL_W), lambda i: (i, 0)),
            pl.BlockSpec((1, HID), lambda i: (0, 0)),
        ],
        out_shape=[
            jax.ShapeDtypeStruct((N, TBL_W), jnp.float32),
            jax.ShapeDtypeStruct((1, HID), jnp.float32),
        ],
    )(x, fin, u_row,
      p["w0"], p["b0"].reshape(1, HID),
      p["w1"], p["b1"].reshape(1, HID),
      p["w2"], p["b2"].reshape(1, HID))


_BIG = 3.4e38
_N_SLAB = C_SEL // K


def _roll(x, d):
    return jnp.concatenate([x[:, d:], x[:, :d]], axis=1)


def _ce_stage(vals, idx, j, desc):
    w = vals.shape[1]
    down_v, up_v = _roll(vals, j), _roll(vals, w - j)
    down_i, up_i = _roll(idx, j), _roll(idx, w - j)
    lane = lax.broadcasted_iota(jnp.int32, vals.shape, 1)
    is_hi = (lane & j) != 0
    pv = jnp.where(is_hi, up_v, down_v)
    pi = jnp.where(is_hi, up_i, down_i)
    want_max = jnp.logical_xor(desc, is_hi)
    keep = (want_max & (vals >= pv)) | (~want_max & (vals <= pv))
    return jnp.where(keep, vals, pv), jnp.where(keep, idx, pi)


def _sort_desc(vals, idx):
    lane = lax.broadcasted_iota(jnp.int32, vals.shape, 1)
    for k in (2, 4, 8, 16, 32, 64, 128):
        desc = (lane & k) == 0
        j = k // 2
        while j >= 1:
            vals, idx = _ce_stage(vals, idx, j, desc)
            j //= 2
    return vals, idx


def _min_upd(m, mg, s, g):
    better = s < m
    return jnp.where(better, s, m), jnp.where(better, g, mg)


def _select_body(xt_ref, a_ref, u_ref, bd_ref, bi_ref, adu_ref, adx_ref,
                 d2_ref):
    step = pl.program_id(0)

    @pl.when(step == 0)
    def _():
        bd_ref[...] = jnp.full((A, K), _BIG, jnp.float32)
        bi_ref[...] = jnp.zeros((A, K), jnp.int32)
        aq = _q(a_ref[...]) * _q(u_ref[...])
        adu_bf = aq[:, 0:1] + aq[:, 1:2] + aq[:, 2:3]
        ae = a_ref[...] * u_ref[...]
        adu_ex = ae[:, 0:1] + ae[:, 1:2] + ae[:, 2:3]
        adu_ref[...] = jnp.broadcast_to(adu_bf, (A, K))
        adx_ref[...] = jnp.broadcast_to(adu_ex, (A, K))

    a0 = a_ref[:, 0:1]
    a1 = a_ref[:, 1:2]
    a2 = a_ref[:, 2:3]
    na2 = a0 * a0 + a1 * a1 + a2 * a2
    a0q, a1q, a2q = _q(a0), _q(a1), _q(a2)

    m = jnp.full((A, K), _BIG, jnp.float32)
    mg = jnp.zeros((A, K), jnp.int32)
    for g in range(_N_SLAB):
        sl = slice(g * K, (g + 1) * K)
        x0 = xt_ref[0:1, sl]
        x1 = xt_ref[1:2, sl]
        x2 = xt_ref[2:3, sl]
        nx2 = x0 * x0 + x1 * x1 + x2 * x2
        dot = a0q * _q(x0) + a1q * _q(x1) + a2q * _q(x2)
        s = jnp.maximum(na2 + nx2 - 2.0 * dot, 0.0)
        d2_ref[:, sl] = s
        m, mg = _min_upd(m, mg, s, g)

    lane = lax.broadcasted_iota(jnp.int32, (A, K), 1)
    base = step * C_SEL
    go0 = jnp.max(jnp.where(m < bd_ref[:, K - 1:K], 1, 0))

    def _cond(carry):
        return carry[0] != 0

    def _body(carry):
        _, m, mg = carry
        gi = base + mg * K + lane
        sv, si = bd_ref[...], bi_ref[...]
        qrow = jnp.sum(jnp.where(m < sv[:, K - 1:K], 1, 0), axis=1,
                       keepdims=True)
        qmax = jnp.max(qrow)

        def _dense(m, gi, sv, si):
            cv, ci = _sort_desc(m, gi)
            take_s = sv <= cv
            mv = jnp.where(take_s, sv, cv)
            mi = jnp.where(take_s, si, ci)
            for j in (64, 32, 16, 8, 4, 2, 1):
                mv, mi = _ce_stage(mv, mi, j, False)
            return mv, mi

        def _sparse(m, gi, sv, si):
            mv, mi, mc = sv, si, m
            for _ in range(4):
                v = jnp.min(mc, axis=1, keepdims=True)
                cid = jnp.min(jnp.where(mc == v, gi, 2147483647), axis=1,
                              keepdims=True)
                do = v < mv[:, K - 1:K]
                rank = jnp.sum(jnp.where(mv < v, 1, 0), axis=1,
                               keepdims=True)
                sh_v = _roll(mv, K - 1)
                sh_i = _roll(mi, K - 1)
                at = lane == rank
                below = lane < rank
                nv = jnp.where(below, mv,
                               jnp.where(at, jnp.broadcast_to(v, (A, K)),
                                         sh_v))
                ni = jnp.where(below, mi,
                               jnp.where(at, jnp.broadcast_to(cid, (A, K)),
                                         sh_i))
                mv = jnp.where(do, nv, mv)
                mi = jnp.where(do, ni, mi)
                mc = jnp.where((mc == v) & (gi == cid), _BIG, mc)
            return mv, mi

        mv, mi = lax.cond(qmax > 4, _dense, _sparse, m, gi, sv, si)
        bd_ref[...] = mv
        bi_ref[...] = mi
        m2 = jnp.full((A, K), _BIG, jnp.float32)
        mg2 = jnp.zeros((A, K), jnp.int32)
        for g in range(_N_SLAB):
            sl = slice(g * K, (g + 1) * K)
            s = jnp.where(mg == g, _BIG, d2_ref[:, sl])
            d2_ref[:, sl] = s
            m2, mg2 = _min_upd(m2, mg2, s, g)
        go2 = jnp.max(jnp.where(m2 < mv[:, K - 1:K], 1, 0))
        return go2, m2, mg2

    lax.while_loop(_cond, _body, (go0, m, mg))


def _run_select(xt, a, u_row):
    grid = N_PAD // C_SEL
    return pl.pallas_call(
        _select_body,
        grid=(grid,),
        in_specs=[
            pl.BlockSpec((3, C_SEL), lambda i: (0, i)),
            pl.BlockSpec((A, 3), lambda i: (0, 0)),
            pl.BlockSpec((1, 3), lambda i: (0, 0)),
        ],
        out_specs=[
            pl.BlockSpec((A, K), lambda i: (0, 0)),
            pl.BlockSpec((A, K), lambda i: (0, 0)),
            pl.BlockSpec((A, K), lambda i: (0, 0)),
            pl.BlockSpec((A, K), lambda i: (0, 0)),
        ],
        out_shape=[
            jax.ShapeDtypeStruct((A, K), jnp.float32),
            jax.ShapeDtypeStruct((A, K), jnp.int32),
            jax.ShapeDtypeStruct((A, K), jnp.float32),
            jax.ShapeDtypeStruct((A, K), jnp.float32),
        ],
        scratch_shapes=[
            pltpu.VMEM((A, C_SEL), jnp.float32),
        ],
    )(xt, a, u_row)


_NC, _NS = 2, 16
_NW = _NC * _NS
_B_TOT = A * K
_B_PER_W = _B_TOT // _NW
_B_CHUNK = 128


def _gather_rows(table, idx_flat):
    mesh = plsc.VectorSubcoreMesh(core_axis_name="c", subcore_axis_name="s")

    @functools.partial(
        pl.kernel,
        out_type=jax.ShapeDtypeStruct((_B_TOT, TBL_W), jnp.float32),
        mesh=mesh,
        scratch_types=[
            pltpu.VMEM((_B_CHUNK,), jnp.int32),
            pltpu.VMEM((_B_CHUNK, TBL_W), jnp.float32),
            pltpu.SemaphoreType.DMA,
        ],
    )
    def gk(table_hbm, idx_hbm, out_hbm, idx_v, rows_v, sem):
        wid = lax.axis_index("s") * _NC + lax.axis_index("c")
        for j in range(_B_PER_W // _B_CHUNK):
            base = wid * _B_PER_W + j * _B_CHUNK
            pltpu.sync_copy(idx_hbm.at[pl.ds(base, _B_CHUNK)], idx_v)
            pltpu.async_copy(table_hbm.at[idx_v], rows_v, sem).wait()
            pltpu.sync_copy(rows_v, out_hbm.at[pl.ds(base, _B_CHUNK)])

    return gk(table, idx_flat)


def _edge_body(g_ref, d2k_ref, adu_ref, adx_ref,
               w0f, w0d2, w0adu, w0ndu, w0rdu, b0, w1, b1, w2, b2,
               gw0, gb0, gw1, gb1,
               eh_ref, lg_ref):
    g = g_ref[...]
    feat = g[:, :HID]
    ndu = g[:, HID:HID + 1]
    ndu_ex = g[:, HID + 1:HID + 2]
    d2k = d2k_ref[...]
    adu = adu_ref[...]
    rdu = adx_ref[...] - ndu_ex
    h = _dot_bf(feat, w0f[...])
    h = h + _q(d2k) * _q(w0d2[...]) + _q(adu) * _q(w0adu[...]) \
        + _q(ndu) * _q(w0ndu[...]) + _q(rdu) * _q(w0rdu[...]) + b0[...]
    h = _gelu(h)
    h = _gelu(_dot_bf(h, w1[...]) + b1[...])
    eh = _dot_bf(h, w2[...]) + b2[...]
    eh_ref[...] = eh
    hg = _gelu(_dot_bf(eh, gw0[...]) + gb0[...])
    lg = _dot_bf(hg, gw1[...]) + gb1[...]
    lg_ref[...] = lg - d2k


def _run_edge(g, d2k_col, adu_col, adx_col, pe, pg):
    grid = _B_TOT // (AB * K)
    rows = AB * K
    full = lambda shape: pl.BlockSpec(shape, lambda i: (0, 0))
    w0 = pe["w0"]
    return pl.pallas_call(
        _edge_body,
        grid=(grid,),
        in_specs=[
            pl.BlockSpec((rows, TBL_W), lambda i: (i, 0)),
            pl.BlockSpec((rows, 1), lambda i: (i, 0)),
            pl.BlockSpec((rows, 1), lambda i: (i, 0)),
            pl.BlockSpec((rows, 1), lambda i: (i, 0)),
            full((HID, HID)), full((1, HID)), full((1, HID)),
            full((1, HID)), full((1, HID)), full((1, HID)),
            full((HID, HID)), full((1, HID)),
            full((HID, HID)), full((1, HID)),
            full((HID, HID)), full((1, HID)),
            full((HID, 1)), full((1, 1)),
        ],
        out_specs=[
            pl.BlockSpec((rows, HID), lambda i: (i, 0)),
            pl.BlockSpec((rows, 1), lambda i: (i, 0)),
        ],
        out_shape=[
            jax.ShapeDtypeStruct((_B_TOT, HID), jnp.float32),
            jax.ShapeDtypeStruct((_B_TOT, 1), jnp.float32),
        ],
    )(g, d2k_col, adu_col, adx_col,
      w0[:HID, :], w0[HID:HID + 1, :], w0[HID + 1:HID + 2, :],
      w0[HID + 2:HID + 3, :], w0[HID + 3:HID + 4, :],
      pe["b0"].reshape(1, HID),
      pe["w1"], pe["b1"].reshape(1, HID),
      pe["w2"], pe["b2"].reshape(1, HID),
      pg["w0"], pg["b0"].reshape(1, HID),
      pg["w1"], pg["b1"].reshape(1, 1))


def _softmax_body(l_ref, w_ref):
    l = l_ref[...]
    m = jnp.max(l, axis=1, keepdims=True)
    e = jnp.exp(l - m)
    w_ref[...] = e / jnp.sum(e, axis=1, keepdims=True)


def _run_softmax(lg):
    return pl.pallas_call(
        _softmax_body,
        in_specs=[pl.BlockSpec((A, K), lambda: (0, 0))],
        out_specs=pl.BlockSpec((A, K), lambda: (0, 0)),
        out_shape=jax.ShapeDtypeStruct((A, K), jnp.float32),
    )(lg)


def _agg_body(eh_ref, wc_ref, fsum_ref,
              gw0, gb0, gw1, gb1, gw2, gb2,
              ow0, ob0, ow1, ob1, ow2, ob2,
              out_ref):
    weh = eh_ref[...] * wc_ref[...]
    rows = weh.shape[0]
    rblk = lax.broadcasted_iota(jnp.int32, (AB, rows), 1) // K
    cblk = lax.broadcasted_iota(jnp.int32, (AB, rows), 0)
    sel = (rblk == cblk).astype(jnp.float32)
    agg = jnp.dot(sel, weh, preferred_element_type=jnp.float32, precision=lax.Precision.HIGHEST)
    mean = fsum_ref[...] * (1.0 / N)
    gc = _mlp3(mean, gw0[...], gb0[...], gw1[...], gb1[...],
               gw2[...], gb2[...])
    gb = jnp.broadcast_to(gc, (AB, HID))
    oi = jnp.concatenate([agg, gb], axis=1)
    out_ref[...] = _mlp3(oi, ow0[...], ob0[...], ow1[...], ob1[...],
                         ow2[...], ob2[...])


def _run_agg(eh, w_col, fsum, pglob, pout):
    grid = A // AB
    rows = AB * K
    full = lambda shape: pl.BlockSpec(shape, lambda i: (0, 0))
    return pl.pallas_call(
        _agg_body,
        grid=(grid,),
        in_specs=[
            pl.BlockSpec((rows, HID), lambda i: (i, 0)),
            pl.BlockSpec((rows, 1), lambda i: (i, 0)),
            full((1, HID)),
            full((HID, HID)), full((1, HID)),
            full((HID, HID)), full((1, HID)),
            full((HID, HID)), full((1, HID)),
            full((2 * HID, HID)), full((1, HID)),
            full((HID, HID)), full((1, HID)),
            full((HID, HID)), full((1, HID)),
        ],
        out_specs=pl.BlockSpec((AB, HID), lambda i: (i, 0)),
        out_shape=jax.ShapeDtypeStruct((A, HID), jnp.float32),
    )(eh, w_col, fsum,
      pglob["w0"], pglob["b0"].reshape(1, HID),
      pglob["w1"], pglob["b1"].reshape(1, HID),
      pglob["w2"], pglob["b2"].reshape(1, HID),
      pout["w0"], pout["b0"].reshape(1, HID),
      pout["w1"], pout["b1"].reshape(1, HID),
      pout["w2"], pout["b2"].reshape(1, HID))


def kernel(input_coords, input_x, anchor_coords, flow_dir, params):
    x = input_coords[0]
    fin = input_x[0]
    a = anchor_coords[0]
    u = flow_dir[0]
    u = u / (jnp.linalg.norm(u) + 1e-08)
    u_row = u.reshape(1, 3)

    table, fsum = _run_prep(x, fin, u_row, params["ne"])
    xt = jnp.pad(x.T, ((0, 0), (0, N_PAD - N)), constant_values=1e15)
    best_d, best_i, adu, adx = _run_select(xt, a, u_row)

    idx_flat = best_i.reshape(_B_TOT)
    g = _gather_rows(table, idx_flat)

    d2k_col = best_d.reshape(_B_TOT, 1)
    adu_col = adu.reshape(_B_TOT, 1)
    adx_col = adx.reshape(_B_TOT, 1)
    eh, lg = _run_edge(g, d2k_col, adu_col, adx_col,
                       params["edge"], params["gate"])

    w = _run_softmax(lg.reshape(A, K))
    w_col = w.reshape(_B_TOT, 1)
    anchor_feat = _run_agg(eh, w_col, fsum, params["glob"], params["out"])
    return anchor_feat[None]
